# baseline (speedup 1.0000x reference)
import functools

import jax
import jax.numpy as jnp
from jax import lax
from jax.experimental import pallas as pl
from jax.experimental.pallas import tpu as pltpu
from jax.experimental.pallas import tpu_sc as plsc

N_SRC = 100000
N_DST = 20000
E = 640000
D = 128
NSC = 2
PER_SC = N_DST // NSC
CHUNK = 160
NCHUNKS = E // CHUNK
NPAIRS = NCHUNKS // 2
NTILES = 16
FCHUNK = 16
NFCHUNKS = PER_SC // FCHUNK


def _agg_body(table, zdst, src_idx, dst_idx, out,
              idx_s, idx_d, idxg0, idxl0, idxg1, idxl1, rows0, rows1, ones_v,
              sbuf, cbuf, obuf, sums_sh, cnt_sh, gsem0, gsem1, ssem0, ssem1):
    c = lax.axis_index("c")
    s = lax.axis_index("s")
    base_dst = c * PER_SC

    zeros16 = jnp.zeros((16,), jnp.float32)
    for j in range(CHUNK // 16):
        ones_v[pl.ds(j * 16, 16)] = jnp.ones((16,), jnp.float32)

    def zrow_init(i, _):
        for j in range(D // 16):
            obuf[i, pl.ds(j * 16, 16)] = zeros16
        return _
    lax.fori_loop(0, FCHUNK, zrow_init, None)

    def cb_init(i, _):
        cbuf[pl.ds(i * 16, 16)] = zeros16
        return _
    lax.fori_loop(0, FCHUNK // 16, cb_init, None)

    nk = (NFCHUNKS - s + NTILES - 1) // NTILES

    def zero_step(t, _):
        r = pl.multiple_of((s + t * NTILES) * FCHUNK, 8)
        pltpu.sync_copy(obuf, sums_sh.at[pl.ds(r, FCHUNK)])
        pltpu.sync_copy(cbuf, cnt_sh.at[pl.ds(r, FCHUNK)])
        return _
    lax.fori_loop(0, nk, zero_step, None)

    plsc.subcore_barrier()

    plo = (s * NPAIRS) // NTILES
    phi = ((s + 1) * NPAIRS) // NTILES

    def stage(i, idxg, idxl):
        base = pl.multiple_of(i * CHUNK, CHUNK)
        pltpu.sync_copy(src_idx.at[pl.ds(base, CHUNK)], idx_s)
        pltpu.sync_copy(dst_idx.at[pl.ds(base, CHUNK)], idx_d)
        for j in range(CHUNK // 16):
            sl = pl.ds(j * 16, 16)
            dl = idx_d[sl] - base_dst
            ok = (dl >= 0) & (dl < PER_SC)
            idxl[sl] = jnp.where(ok, dl, -1)
            idxg[sl] = jnp.where(ok, idx_s[sl], -1)

    def scatter(rows, idxl):
        pltpu.sync_copy(rows, sums_sh.at[plsc.Indices(idxl, ignored_value=-1)],
                        add=True)
        pltpu.sync_copy(ones_v, cnt_sh.at[plsc.Indices(idxl, ignored_value=-1)],
                        add=True)

    def gather_start(idxg, rows, gsem):
        pltpu.async_copy(
            table.at[plsc.Indices(idxg, ignored_value=-1)], rows, gsem)

    def gather_wait(idxg, rows, gsem):
        pltpu.make_async_copy(
            table.at[plsc.Indices(idxg, ignored_value=-1)], rows, gsem).wait()

    def scatter_start(rows, idxl, ssem):
        pltpu.async_copy(
            rows, sums_sh.at[plsc.Indices(idxl, ignored_value=-1)], ssem,
            add=True)
        pltpu.async_copy(
            ones_v, cnt_sh.at[plsc.Indices(idxl, ignored_value=-1)], ssem,
            add=True)

    def scatter_wait(rows, idxl, ssem):
        pltpu.make_async_copy(
            rows, sums_sh.at[plsc.Indices(idxl, ignored_value=-1)],
            ssem).wait()
        pltpu.make_async_copy(
            ones_v, cnt_sh.at[plsc.Indices(idxl, ignored_value=-1)],
            ssem).wait()

    stage(2 * plo, idxg0, idxl0)
    gather_start(idxg0, rows0, gsem0)

    def pair_step(t, _):
        i0 = 2 * t

        @pl.when(t > plo)
        def _():
            scatter_wait(rows1, idxl1, ssem1)

        stage(i0 + 1, idxg1, idxl1)
        gather_wait(idxg0, rows0, gsem0)
        gather_start(idxg1, rows1, gsem1)
        scatter_start(rows0, idxl0, ssem0)
        gather_wait(idxg1, rows1, gsem1)
        scatter_wait(rows0, idxl0, ssem0)

        @pl.when(t + 1 < phi)
        def _():
            stage(i0 + 2, idxg0, idxl0)
            gather_start(idxg0, rows0, gsem0)
        scatter_start(rows1, idxl1, ssem1)
        return _

    lax.fori_loop(plo, phi, pair_step, None)
    scatter_wait(rows1, idxl1, ssem1)

    plsc.subcore_barrier()

    def fin_step(t, _):
        rl = pl.multiple_of((s + t * NTILES) * FCHUNK, 8)
        rg = pl.multiple_of(base_dst + rl, 8)
        pltpu.sync_copy(sums_sh.at[pl.ds(rl, FCHUNK)], sbuf)
        pltpu.sync_copy(cnt_sh.at[pl.ds(rl, FCHUNK)], cbuf)
        pltpu.sync_copy(zdst.at[pl.ds(rg, FCHUNK)], obuf)

        def row_block(b, _):
            i0 = pl.multiple_of(b * 16, 16)
            inv16 = 0.3 / jnp.maximum(cbuf[pl.ds(i0, 16)], 1.0)
            for lane in range(16):
                i = i0 + lane
                inv = inv16[lane]
                for j in range(D // 16):
                    sl = pl.ds(j * 16, 16)
                    obuf[i, sl] = obuf[i, sl] * 0.7 + sbuf[i, sl] * inv
            return _
        lax.fori_loop(0, FCHUNK // 16, row_block, None)
        pltpu.sync_copy(obuf, out.at[pl.ds(rg, FCHUNK)])
        return _

    lax.fori_loop(0, nk, fin_step, None)


@functools.partial(
    pl.kernel,
    out_type=jax.ShapeDtypeStruct((N_DST, D), jnp.float32),
    mesh=plsc.VectorSubcoreMesh(core_axis_name="c", subcore_axis_name="s"),
    scratch_types=[
        pltpu.VMEM((CHUNK,), jnp.int32),
        pltpu.VMEM((CHUNK,), jnp.int32),
        pltpu.VMEM((CHUNK,), jnp.int32),
        pltpu.VMEM((CHUNK,), jnp.int32),
        pltpu.VMEM((CHUNK,), jnp.int32),
        pltpu.VMEM((CHUNK,), jnp.int32),
        pltpu.VMEM((CHUNK, D), jnp.float32),
        pltpu.VMEM((CHUNK, D), jnp.float32),
        pltpu.VMEM((CHUNK,), jnp.float32),
        pltpu.VMEM((FCHUNK, D), jnp.float32),
        pltpu.VMEM((FCHUNK,), jnp.float32),
        pltpu.VMEM((FCHUNK, D), jnp.float32),
        pltpu.VMEM_SHARED((PER_SC, D), jnp.float32),
        pltpu.VMEM_SHARED((PER_SC,), jnp.float32),
        pltpu.SemaphoreType.DMA,
        pltpu.SemaphoreType.DMA,
        pltpu.SemaphoreType.DMA,
        pltpu.SemaphoreType.DMA,
    ],
)
def _agg_call(table, zdst, src_idx, dst_idx, out, *scratch):
    _agg_body(table, zdst, src_idx, dst_idx, out, *scratch)


@jax.jit
def _run(z_bill_version, z_bill, z_legislator_term, z_legislator,
         src_is_version, dst_is_version, src_same_person, dst_same_person):
    out_b = _agg_call(z_bill_version, z_bill, src_is_version, dst_is_version)
    out_l = _agg_call(z_legislator_term, z_legislator,
                      src_same_person, dst_same_person)
    return out_b, out_l


def kernel(z_bill_version, z_bill, z_legislator_term, z_legislator,
           src_is_version, dst_is_version, src_same_person, dst_same_person):
    return _run(z_bill_version, z_bill, z_legislator_term, z_legislator,
                src_is_version, dst_is_version, src_same_person,
                dst_same_person)


# speedup vs baseline: 1.1964x; 1.1964x over previous
import functools

import jax
import jax.numpy as jnp
from jax import lax
from jax.experimental import pallas as pl
from jax.experimental.pallas import tpu as pltpu
from jax.experimental.pallas import tpu_sc as plsc

N_SRC = 100000
N_DST = 20000
E = 640000
D = 128
NSC = 2
PER_SC = N_DST // NSC
CHUNK = 160
NCHUNKS = E // CHUNK
NPAIRS = NCHUNKS // 2
NTILES = 16
FCHUNK = 16
NFCHUNKS = PER_SC // FCHUNK


def _agg_body(table, src_idx, dst_idx, sums_out, cnt_out,
              idx_s, idx_d, idxg0, idxl0, idxg1, idxl1, rows0, rows1, ones_v,
              cbuf, obuf, cnt_vm, sums_sh, cnt_sh, gsem0, gsem1, ssem0,
              ssem1):
    c = lax.axis_index("c")
    s = lax.axis_index("s")
    base_dst = c * PER_SC

    zeros16 = jnp.zeros((16,), jnp.float32)
    for j in range(CHUNK // 16):
        ones_v[pl.ds(j * 16, 16)] = jnp.ones((16,), jnp.float32)

    def zrow_init(i, _):
        for j in range(D // 16):
            obuf[i, pl.ds(j * 16, 16)] = zeros16
        return _
    lax.fori_loop(0, FCHUNK, zrow_init, None)

    def cb_init(i, _):
        cbuf[pl.ds(i * 16, 16)] = zeros16
        return _
    lax.fori_loop(0, FCHUNK // 16, cb_init, None)

    nk = (NFCHUNKS - s + NTILES - 1) // NTILES

    def zero_step(t, _):
        r = pl.multiple_of((s + t * NTILES) * FCHUNK, 8)
        pltpu.sync_copy(obuf, sums_sh.at[pl.ds(r, FCHUNK)])
        pltpu.sync_copy(cbuf, cnt_sh.at[pl.ds(r, FCHUNK)])
        return _
    lax.fori_loop(0, nk, zero_step, None)

    plsc.subcore_barrier()

    plo = (s * NPAIRS) // NTILES
    phi = ((s + 1) * NPAIRS) // NTILES

    def stage(i, idxg, idxl):
        base = pl.multiple_of(i * CHUNK, CHUNK)
        pltpu.sync_copy(src_idx.at[pl.ds(base, CHUNK)], idx_s)
        pltpu.sync_copy(dst_idx.at[pl.ds(base, CHUNK)], idx_d)
        for j in range(CHUNK // 16):
            sl = pl.ds(j * 16, 16)
            dl = idx_d[sl] - base_dst
            ok = (dl >= 0) & (dl < PER_SC)
            idxl[sl] = jnp.where(ok, dl, -1)
            idxg[sl] = jnp.where(ok, idx_s[sl], -1)

    def scatter(rows, idxl):
        pltpu.sync_copy(rows, sums_sh.at[plsc.Indices(idxl, ignored_value=-1)],
                        add=True)
        pltpu.sync_copy(ones_v, cnt_sh.at[plsc.Indices(idxl, ignored_value=-1)],
                        add=True)

    def gather_start(idxg, rows, gsem):
        pltpu.async_copy(
            table.at[plsc.Indices(idxg, ignored_value=-1)], rows, gsem)

    def gather_wait(idxg, rows, gsem):
        pltpu.make_async_copy(
            table.at[plsc.Indices(idxg, ignored_value=-1)], rows, gsem).wait()

    def scatter_start(rows, idxl, ssem):
        pltpu.async_copy(
            rows, sums_sh.at[plsc.Indices(idxl, ignored_value=-1)], ssem,
            add=True)
        pltpu.async_copy(
            ones_v, cnt_sh.at[plsc.Indices(idxl, ignored_value=-1)], ssem,
            add=True)

    def scatter_wait(rows, idxl, ssem):
        pltpu.make_async_copy(
            rows, sums_sh.at[plsc.Indices(idxl, ignored_value=-1)],
            ssem).wait()
        pltpu.make_async_copy(
            ones_v, cnt_sh.at[plsc.Indices(idxl, ignored_value=-1)],
            ssem).wait()

    stage(2 * plo, idxg0, idxl0)
    gather_start(idxg0, rows0, gsem0)

    def pair_step(t, _):
        i0 = 2 * t

        @pl.when(t > plo)
        def _():
            scatter_wait(rows1, idxl1, ssem1)

        stage(i0 + 1, idxg1, idxl1)
        gather_wait(idxg0, rows0, gsem0)
        gather_start(idxg1, rows1, gsem1)
        scatter_start(rows0, idxl0, ssem0)
        scatter_wait(rows0, idxl0, ssem0)

        @pl.when(t + 1 < phi)
        def _():
            stage(i0 + 2, idxg0, idxl0)
            gather_start(idxg0, rows0, gsem0)
        gather_wait(idxg1, rows1, gsem1)
        scatter_start(rows1, idxl1, ssem1)
        return _

    lax.fori_loop(plo, phi, pair_step, None)
    scatter_wait(rows1, idxl1, ssem1)

    plsc.subcore_barrier()

    DUMP = 624
    r0 = pl.multiple_of(s * DUMP, 8)
    g0 = pl.multiple_of(base_dst + r0, 8)
    pltpu.sync_copy(sums_sh.at[pl.ds(r0, DUMP)],
                    sums_out.at[pl.ds(g0, DUMP)])
    pltpu.sync_copy(cnt_sh.at[pl.ds(r0, DUMP)], cnt_vm)
    pltpu.sync_copy(cnt_vm, cnt_out.at[pl.ds(g0, DUMP)])

    @pl.when(s == 0)
    def _tail():
        rt = pl.multiple_of(NTILES * DUMP, 8)
        gt = pl.multiple_of(base_dst + rt, 8)
        pltpu.sync_copy(sums_sh.at[pl.ds(rt, PER_SC - NTILES * DUMP)],
                        sums_out.at[pl.ds(gt, PER_SC - NTILES * DUMP)])
        pltpu.sync_copy(cnt_sh.at[pl.ds(rt, PER_SC - NTILES * DUMP)],
                        cnt_vm.at[pl.ds(0, PER_SC - NTILES * DUMP)])
        pltpu.sync_copy(cnt_vm.at[pl.ds(0, PER_SC - NTILES * DUMP)],
                        cnt_out.at[pl.ds(gt, PER_SC - NTILES * DUMP)])


@functools.partial(
    pl.kernel,
    out_type=(jax.ShapeDtypeStruct((N_DST, D), jnp.float32),
              jax.ShapeDtypeStruct((N_DST,), jnp.float32)),
    mesh=plsc.VectorSubcoreMesh(core_axis_name="c", subcore_axis_name="s"),
    scratch_types=[
        pltpu.VMEM((CHUNK,), jnp.int32),
        pltpu.VMEM((CHUNK,), jnp.int32),
        pltpu.VMEM((CHUNK,), jnp.int32),
        pltpu.VMEM((CHUNK,), jnp.int32),
        pltpu.VMEM((CHUNK,), jnp.int32),
        pltpu.VMEM((CHUNK,), jnp.int32),
        pltpu.VMEM((CHUNK, D), jnp.float32),
        pltpu.VMEM((CHUNK, D), jnp.float32),
        pltpu.VMEM((CHUNK,), jnp.float32),
        pltpu.VMEM((FCHUNK,), jnp.float32),
        pltpu.VMEM((FCHUNK, D), jnp.float32),
        pltpu.VMEM((624,), jnp.float32),
        pltpu.VMEM_SHARED((PER_SC, D), jnp.float32),
        pltpu.VMEM_SHARED((PER_SC,), jnp.float32),
        pltpu.SemaphoreType.DMA,
        pltpu.SemaphoreType.DMA,
        pltpu.SemaphoreType.DMA,
        pltpu.SemaphoreType.DMA,
    ],
)
def _agg_call(table, src_idx, dst_idx, sums_out, cnt_out, *scratch):
    _agg_body(table, src_idx, dst_idx, sums_out, cnt_out, *scratch)


def _blend_body(z_ref, s_ref, c_ref, o_ref):
    o_ref[...] = (0.7 * z_ref[...]
                  + 0.3 * s_ref[...] / jnp.maximum(c_ref[...], 1.0))


BLK = 2000


def _blend(z, sums, cnt):
    return pl.pallas_call(
        _blend_body,
        out_shape=jax.ShapeDtypeStruct((N_DST, D), jnp.float32),
        grid=(N_DST // BLK,),
        in_specs=[
            pl.BlockSpec((BLK, D), lambda i: (i, 0)),
            pl.BlockSpec((BLK, D), lambda i: (i, 0)),
            pl.BlockSpec((BLK, 1), lambda i: (i, 0)),
        ],
        out_specs=pl.BlockSpec((BLK, D), lambda i: (i, 0)),
    )(z, sums, cnt.reshape(N_DST, 1))


@jax.jit
def _run(z_bill_version, z_bill, z_legislator_term, z_legislator,
         src_is_version, dst_is_version, src_same_person, dst_same_person):
    sums_b, cnt_b = _agg_call(z_bill_version, src_is_version, dst_is_version)
    sums_l, cnt_l = _agg_call(z_legislator_term,
                              src_same_person, dst_same_person)
    out_b = _blend(z_bill, sums_b, cnt_b)
    out_l = _blend(z_legislator, sums_l, cnt_l)
    return out_b, out_l


def kernel(z_bill_version, z_bill, z_legislator_term, z_legislator,
           src_is_version, dst_is_version, src_same_person, dst_same_person):
    return _run(z_bill_version, z_bill, z_legislator_term, z_legislator,
                src_is_version, dst_is_version, src_same_person,
                dst_same_person)


# speedup vs baseline: 1.2053x; 1.0075x over previous
import functools

import jax
import jax.numpy as jnp
from jax import lax
from jax.experimental import pallas as pl
from jax.experimental.pallas import tpu as pltpu
from jax.experimental.pallas import tpu_sc as plsc

N_SRC = 100000
N_DST = 20000
E = 640000
D = 128
NSC = 2
PER_SC = N_DST // NSC
CHUNK = 160
NCHUNKS = E // CHUNK
NPAIRS = NCHUNKS // 2
NTILES = 16
FCHUNK = 40
NFCHUNKS = PER_SC // FCHUNK


def _agg_body(table, src_idx, dst_idx, sums_out, cnt_out,
              idx_s, idx_d, idxg0, idxl0, idxg1, idxl1, rows0, rows1, ones_v,
              cbuf, obuf, cnt_vm, sums_sh, cnt_sh, gsem0, gsem1, ssem0,
              ssem1):
    c = lax.axis_index("c")
    s = lax.axis_index("s")
    base_dst = c * PER_SC

    zeros16 = jnp.zeros((16,), jnp.float32)
    for j in range(CHUNK // 16):
        ones_v[pl.ds(j * 16, 16)] = jnp.ones((16,), jnp.float32)

    def zrow_init(i, _):
        for j in range(D // 16):
            obuf[i, pl.ds(j * 16, 16)] = zeros16
        return _
    lax.fori_loop(0, FCHUNK, zrow_init, None)

    def cb_init(i, _):
        cbuf[pl.ds(i * 16, 16)] = zeros16
        return _
    lax.fori_loop(0, FCHUNK // 16, cb_init, None)
    if FCHUNK % 16:
        cbuf[pl.ds(FCHUNK - 16, 16)] = zeros16

    nk = (NFCHUNKS - s + NTILES - 1) // NTILES

    def zero_fire(t):
        r = pl.multiple_of((s + t * NTILES) * FCHUNK, 8)
        pltpu.async_copy(obuf, sums_sh.at[pl.ds(r, FCHUNK)], gsem0)
        pltpu.async_copy(cbuf, cnt_sh.at[pl.ds(r, FCHUNK)], gsem1)

    def zero_wait(t):
        r = pl.multiple_of((s + t * NTILES) * FCHUNK, 8)
        pltpu.make_async_copy(obuf, sums_sh.at[pl.ds(r, FCHUNK)],
                              gsem0).wait()
        pltpu.make_async_copy(cbuf, cnt_sh.at[pl.ds(r, FCHUNK)],
                              gsem1).wait()

    zero_fire(0)

    def zero_step(t, _):
        @pl.when(t + 1 < nk)
        def _():
            zero_fire(t + 1)
        zero_wait(t)
        return _
    lax.fori_loop(0, nk, zero_step, None)

    plsc.subcore_barrier()

    plo = (s * NPAIRS) // NTILES
    phi = ((s + 1) * NPAIRS) // NTILES

    def stage(i, idxg, idxl):
        base = pl.multiple_of(i * CHUNK, CHUNK)
        pltpu.sync_copy(src_idx.at[pl.ds(base, CHUNK)], idx_s)
        pltpu.sync_copy(dst_idx.at[pl.ds(base, CHUNK)], idx_d)
        for j in range(CHUNK // 16):
            sl = pl.ds(j * 16, 16)
            dl = idx_d[sl] - base_dst
            ok = (dl >= 0) & (dl < PER_SC)
            idxl[sl] = jnp.where(ok, dl, -1)
            idxg[sl] = jnp.where(ok, idx_s[sl], -1)

    def scatter(rows, idxl):
        pltpu.sync_copy(rows, sums_sh.at[plsc.Indices(idxl, ignored_value=-1)],
                        add=True)
        pltpu.sync_copy(ones_v, cnt_sh.at[plsc.Indices(idxl, ignored_value=-1)],
                        add=True)

    def gather_start(idxg, rows, gsem):
        pltpu.async_copy(
            table.at[plsc.Indices(idxg, ignored_value=-1)], rows, gsem)

    def gather_wait(idxg, rows, gsem):
        pltpu.make_async_copy(
            table.at[plsc.Indices(idxg, ignored_value=-1)], rows, gsem).wait()

    def scatter_start(rows, idxl, ssem):
        pltpu.async_copy(
            rows, sums_sh.at[plsc.Indices(idxl, ignored_value=-1)], ssem,
            add=True)
        pltpu.async_copy(
            ones_v, cnt_sh.at[plsc.Indices(idxl, ignored_value=-1)], ssem,
            add=True)

    def scatter_wait(rows, idxl, ssem):
        pltpu.make_async_copy(
            rows, sums_sh.at[plsc.Indices(idxl, ignored_value=-1)],
            ssem).wait()
        pltpu.make_async_copy(
            ones_v, cnt_sh.at[plsc.Indices(idxl, ignored_value=-1)],
            ssem).wait()

    stage(2 * plo, idxg0, idxl0)
    gather_start(idxg0, rows0, gsem0)

    def pair_step(t, _):
        i0 = 2 * t

        @pl.when(t > plo)
        def _():
            scatter_wait(rows1, idxl1, ssem1)

        stage(i0 + 1, idxg1, idxl1)
        gather_wait(idxg0, rows0, gsem0)
        gather_start(idxg1, rows1, gsem1)
        scatter_start(rows0, idxl0, ssem0)
        scatter_wait(rows0, idxl0, ssem0)

        @pl.when(t + 1 < phi)
        def _():
            stage(i0 + 2, idxg0, idxl0)
            gather_start(idxg0, rows0, gsem0)
        gather_wait(idxg1, rows1, gsem1)
        scatter_start(rows1, idxl1, ssem1)
        return _

    lax.fori_loop(plo, phi, pair_step, None)
    scatter_wait(rows1, idxl1, ssem1)

    plsc.subcore_barrier()

    DUMP = 624
    r0 = pl.multiple_of(s * DUMP, 8)
    g0 = pl.multiple_of(base_dst + r0, 8)
    pltpu.sync_copy(sums_sh.at[pl.ds(r0, DUMP)],
                    sums_out.at[pl.ds(g0, DUMP)])
    pltpu.sync_copy(cnt_sh.at[pl.ds(r0, DUMP)], cnt_vm)
    pltpu.sync_copy(cnt_vm, cnt_out.at[pl.ds(g0, DUMP)])

    @pl.when(s == 0)
    def _tail():
        rt = pl.multiple_of(NTILES * DUMP, 8)
        gt = pl.multiple_of(base_dst + rt, 8)
        pltpu.sync_copy(sums_sh.at[pl.ds(rt, PER_SC - NTILES * DUMP)],
                        sums_out.at[pl.ds(gt, PER_SC - NTILES * DUMP)])
        pltpu.sync_copy(cnt_sh.at[pl.ds(rt, PER_SC - NTILES * DUMP)],
                        cnt_vm.at[pl.ds(0, PER_SC - NTILES * DUMP)])
        pltpu.sync_copy(cnt_vm.at[pl.ds(0, PER_SC - NTILES * DUMP)],
                        cnt_out.at[pl.ds(gt, PER_SC - NTILES * DUMP)])


@functools.partial(
    pl.kernel,
    out_type=(jax.ShapeDtypeStruct((N_DST, D), jnp.float32),
              jax.ShapeDtypeStruct((N_DST,), jnp.float32)),
    mesh=plsc.VectorSubcoreMesh(core_axis_name="c", subcore_axis_name="s"),
    scratch_types=[
        pltpu.VMEM((CHUNK,), jnp.int32),
        pltpu.VMEM((CHUNK,), jnp.int32),
        pltpu.VMEM((CHUNK,), jnp.int32),
        pltpu.VMEM((CHUNK,), jnp.int32),
        pltpu.VMEM((CHUNK,), jnp.int32),
        pltpu.VMEM((CHUNK,), jnp.int32),
        pltpu.VMEM((CHUNK, D), jnp.float32),
        pltpu.VMEM((CHUNK, D), jnp.float32),
        pltpu.VMEM((CHUNK,), jnp.float32),
        pltpu.VMEM((FCHUNK,), jnp.float32),
        pltpu.VMEM((FCHUNK, D), jnp.float32),
        pltpu.VMEM((624,), jnp.float32),
        pltpu.VMEM_SHARED((PER_SC, D), jnp.float32),
        pltpu.VMEM_SHARED((PER_SC,), jnp.float32),
        pltpu.SemaphoreType.DMA,
        pltpu.SemaphoreType.DMA,
        pltpu.SemaphoreType.DMA,
        pltpu.SemaphoreType.DMA,
    ],
)
def _agg_call(table, src_idx, dst_idx, sums_out, cnt_out, *scratch):
    _agg_body(table, src_idx, dst_idx, sums_out, cnt_out, *scratch)


def _blend_body(z_ref, s_ref, c_ref, o_ref):
    o_ref[...] = (0.7 * z_ref[...]
                  + 0.3 * s_ref[...] / jnp.maximum(c_ref[...], 1.0))


BLK = 2000


def _blend(z, sums, cnt):
    return pl.pallas_call(
        _blend_body,
        out_shape=jax.ShapeDtypeStruct((N_DST, D), jnp.float32),
        grid=(N_DST // BLK,),
        in_specs=[
            pl.BlockSpec((BLK, D), lambda i: (i, 0)),
            pl.BlockSpec((BLK, D), lambda i: (i, 0)),
            pl.BlockSpec((BLK, 1), lambda i: (i, 0)),
        ],
        out_specs=pl.BlockSpec((BLK, D), lambda i: (i, 0)),
    )(z, sums, cnt.reshape(N_DST, 1))


@jax.jit
def _run(z_bill_version, z_bill, z_legislator_term, z_legislator,
         src_is_version, dst_is_version, src_same_person, dst_same_person):
    sums_b, cnt_b = _agg_call(z_bill_version, src_is_version, dst_is_version)
    sums_l, cnt_l = _agg_call(z_legislator_term,
                              src_same_person, dst_same_person)
    out_b = _blend(z_bill, sums_b, cnt_b)
    out_l = _blend(z_legislator, sums_l, cnt_l)
    return out_b, out_l


def kernel(z_bill_version, z_bill, z_legislator_term, z_legislator,
           src_is_version, dst_is_version, src_same_person, dst_same_person):
    return _run(z_bill_version, z_bill, z_legislator_term, z_legislator,
                src_is_version, dst_is_version, src_same_person,
                dst_same_person)


# speedup vs baseline: 1.3678x; 1.1348x over previous
import functools

import jax
import jax.numpy as jnp
from jax import lax
from jax.experimental import pallas as pl
from jax.experimental.pallas import tpu as pltpu
from jax.experimental.pallas import tpu_sc as plsc

N_SRC = 100000
N_DST = 20000
E = 640000
D = 128
NSC = 2
PER_SC = N_DST // NSC
CHUNK = 128
NCHUNKS = E // CHUNK
NPAIRS = NCHUNKS // 2
NTILES = 16
FCHUNK = 40
NFCHUNKS = PER_SC // FCHUNK


def _agg_body(table, sd_idx, sums_out, cnt_out,
              sidx, idxg0, idxl0, idxg1, idxl1, rows0, rows1, ones_v,
              cbuf, obuf, cnt_vm, sums_sh, cnt_sh, gsem0, gsem1, ssem0,
              ssem1):
    c = lax.axis_index("c")
    s = lax.axis_index("s")
    base_dst = c * PER_SC

    zeros16 = jnp.zeros((16,), jnp.float32)
    for j in range(CHUNK // 16):
        ones_v[pl.ds(j * 16, 16)] = jnp.ones((16,), jnp.float32)

    def zrow_init(i, _):
        for j in range(D // 16):
            obuf[i, pl.ds(j * 16, 16)] = zeros16
        return _
    lax.fori_loop(0, FCHUNK, zrow_init, None)

    def cb_init(i, _):
        cbuf[pl.ds(i * 16, 16)] = zeros16
        return _
    lax.fori_loop(0, FCHUNK // 16, cb_init, None)
    if FCHUNK % 16:
        cbuf[pl.ds(FCHUNK - 16, 16)] = zeros16

    nk = (NFCHUNKS - s + NTILES - 1) // NTILES

    def zero_fire(t):
        r = pl.multiple_of((s + t * NTILES) * FCHUNK, 8)
        pltpu.async_copy(obuf, sums_sh.at[pl.ds(r, FCHUNK)], gsem0)
        pltpu.async_copy(cbuf, cnt_sh.at[pl.ds(r, FCHUNK)], gsem1)

    def zero_wait(t):
        r = pl.multiple_of((s + t * NTILES) * FCHUNK, 8)
        pltpu.make_async_copy(obuf, sums_sh.at[pl.ds(r, FCHUNK)],
                              gsem0).wait()
        pltpu.make_async_copy(cbuf, cnt_sh.at[pl.ds(r, FCHUNK)],
                              gsem1).wait()

    zero_fire(0)

    def zero_step(t, _):
        @pl.when(t + 1 < nk)
        def _():
            zero_fire(t + 1)
        zero_wait(t)
        return _
    lax.fori_loop(0, nk, zero_step, None)

    plsc.subcore_barrier()

    plo = (s * NPAIRS) // NTILES
    phi = ((s + 1) * NPAIRS) // NTILES

    def sidx_load(t):
        base = pl.multiple_of(t * (2 * CHUNK), 2 * CHUNK)
        pltpu.sync_copy(sd_idx.at[:, pl.ds(base, 2 * CHUNK)], sidx)

    def compute_idx(half, idxg, idxl):
        for j in range(CHUNK // 16):
            sl16 = pl.ds(half * CHUNK + j * 16, 16)
            sl = pl.ds(j * 16, 16)
            dl = sidx[1, sl16] - base_dst
            ok = (dl >= 0) & (dl < PER_SC)
            idxl[sl] = jnp.where(ok, dl, -1)
            idxg[sl] = jnp.where(ok, sidx[0, sl16], -1)

    def scatter(rows, idxl):
        pltpu.sync_copy(rows, sums_sh.at[plsc.Indices(idxl, ignored_value=-1)],
                        add=True)
        pltpu.sync_copy(ones_v, cnt_sh.at[plsc.Indices(idxl, ignored_value=-1)],
                        add=True)

    def gather_start(idxg, rows, gsem):
        pltpu.async_copy(
            table.at[plsc.Indices(idxg, ignored_value=-1)], rows, gsem)

    def gather_wait(idxg, rows, gsem):
        pltpu.make_async_copy(
            table.at[plsc.Indices(idxg, ignored_value=-1)], rows, gsem).wait()

    def scatter_start(rows, idxl, ssem):
        pltpu.async_copy(
            rows, sums_sh.at[plsc.Indices(idxl, ignored_value=-1)], ssem,
            add=True)
        pltpu.async_copy(
            ones_v, cnt_sh.at[plsc.Indices(idxl, ignored_value=-1)], ssem,
            add=True)

    def scatter_wait(rows, idxl, ssem):
        pltpu.make_async_copy(
            rows, sums_sh.at[plsc.Indices(idxl, ignored_value=-1)],
            ssem).wait()
        pltpu.make_async_copy(
            ones_v, cnt_sh.at[plsc.Indices(idxl, ignored_value=-1)],
            ssem).wait()

    sidx_load(plo)
    compute_idx(0, idxg0, idxl0)
    compute_idx(1, idxg1, idxl1)
    gather_start(idxg0, rows0, gsem0)

    def pair_step(t, _):
        @pl.when(t > plo)
        def _():
            scatter_wait(rows1, idxl1, ssem1)
            compute_idx(1, idxg1, idxl1)

        gather_wait(idxg0, rows0, gsem0)
        gather_start(idxg1, rows1, gsem1)
        scatter_start(rows0, idxl0, ssem0)
        scatter_wait(rows0, idxl0, ssem0)

        @pl.when(t + 1 < phi)
        def _():
            sidx_load(t + 1)
            compute_idx(0, idxg0, idxl0)
            gather_start(idxg0, rows0, gsem0)
        gather_wait(idxg1, rows1, gsem1)
        scatter_start(rows1, idxl1, ssem1)
        return _

    lax.fori_loop(plo, phi, pair_step, None)
    scatter_wait(rows1, idxl1, ssem1)

    plsc.subcore_barrier()

    DUMP = 624
    r0 = pl.multiple_of(s * DUMP, 8)
    g0 = pl.multiple_of(base_dst + r0, 8)
    pltpu.sync_copy(sums_sh.at[pl.ds(r0, DUMP)],
                    sums_out.at[pl.ds(g0, DUMP)])
    pltpu.sync_copy(cnt_sh.at[pl.ds(r0, DUMP)], cnt_vm)
    pltpu.sync_copy(cnt_vm, cnt_out.at[pl.ds(g0, DUMP)])

    @pl.when(s == 0)
    def _tail():
        rt = pl.multiple_of(NTILES * DUMP, 8)
        gt = pl.multiple_of(base_dst + rt, 8)
        pltpu.sync_copy(sums_sh.at[pl.ds(rt, PER_SC - NTILES * DUMP)],
                        sums_out.at[pl.ds(gt, PER_SC - NTILES * DUMP)])
        pltpu.sync_copy(cnt_sh.at[pl.ds(rt, PER_SC - NTILES * DUMP)],
                        cnt_vm.at[pl.ds(0, PER_SC - NTILES * DUMP)])
        pltpu.sync_copy(cnt_vm.at[pl.ds(0, PER_SC - NTILES * DUMP)],
                        cnt_out.at[pl.ds(gt, PER_SC - NTILES * DUMP)])


@functools.partial(
    pl.kernel,
    out_type=(jax.ShapeDtypeStruct((N_DST, D), jnp.float32),
              jax.ShapeDtypeStruct((N_DST,), jnp.float32)),
    mesh=plsc.VectorSubcoreMesh(core_axis_name="c", subcore_axis_name="s"),
    scratch_types=[
        pltpu.VMEM((2, 2 * CHUNK), jnp.int32),
        pltpu.VMEM((CHUNK,), jnp.int32),
        pltpu.VMEM((CHUNK,), jnp.int32),
        pltpu.VMEM((CHUNK,), jnp.int32),
        pltpu.VMEM((CHUNK,), jnp.int32),
        pltpu.VMEM((CHUNK, D), jnp.float32),
        pltpu.VMEM((CHUNK, D), jnp.float32),
        pltpu.VMEM((CHUNK,), jnp.float32),
        pltpu.VMEM((FCHUNK,), jnp.float32),
        pltpu.VMEM((FCHUNK, D), jnp.float32),
        pltpu.VMEM((624,), jnp.float32),
        pltpu.VMEM_SHARED((PER_SC, D), jnp.float32),
        pltpu.VMEM_SHARED((PER_SC,), jnp.float32),
        pltpu.SemaphoreType.DMA,
        pltpu.SemaphoreType.DMA,
        pltpu.SemaphoreType.DMA,
        pltpu.SemaphoreType.DMA,
    ],
)
def _agg_call(table, sd_idx, sums_out, cnt_out, *scratch):
    _agg_body(table, sd_idx, sums_out, cnt_out, *scratch)


def _blend_body(z_ref, s_ref, c_ref, o_ref):
    o_ref[...] = (0.7 * z_ref[...]
                  + 0.3 * s_ref[...] / jnp.maximum(c_ref[...], 1.0))


BLK = 2000


def _blend(z, sums, cnt):
    return pl.pallas_call(
        _blend_body,
        out_shape=jax.ShapeDtypeStruct((N_DST, D), jnp.float32),
        grid=(N_DST // BLK,),
        in_specs=[
            pl.BlockSpec((BLK, D), lambda i: (i, 0)),
            pl.BlockSpec((BLK, D), lambda i: (i, 0)),
            pl.BlockSpec((BLK, 1), lambda i: (i, 0)),
        ],
        out_specs=pl.BlockSpec((BLK, D), lambda i: (i, 0)),
    )(z, sums, cnt.reshape(N_DST, 1))


@jax.jit
def _run(z_bill_version, z_bill, z_legislator_term, z_legislator,
         src_is_version, dst_is_version, src_same_person, dst_same_person):
    sd_b = jnp.stack([src_is_version, dst_is_version])
    sd_l = jnp.stack([src_same_person, dst_same_person])
    sums_b, cnt_b = _agg_call(z_bill_version, sd_b)
    sums_l, cnt_l = _agg_call(z_legislator_term, sd_l)
    out_b = _blend(z_bill, sums_b, cnt_b)
    out_l = _blend(z_legislator, sums_l, cnt_l)
    return out_b, out_l


def kernel(z_bill_version, z_bill, z_legislator_term, z_legislator,
           src_is_version, dst_is_version, src_same_person, dst_same_person):
    return _run(z_bill_version, z_bill, z_legislator_term, z_legislator,
                src_is_version, dst_is_version, src_same_person,
                dst_same_person)


# speedup vs baseline: 1.3806x; 1.0094x over previous
import functools

import jax
import jax.numpy as jnp
from jax import lax
from jax.experimental import pallas as pl
from jax.experimental.pallas import tpu as pltpu
from jax.experimental.pallas import tpu_sc as plsc

N_SRC = 100000
N_DST = 20000
E = 640000
D = 128
NSC = 2
PER_SC = N_DST // NSC
CHUNK = 128
NCHUNKS = E // CHUNK
NPAIRS = NCHUNKS // 2
NTILES = 16
FCHUNK = 40
NFCHUNKS = PER_SC // FCHUNK


def _agg_body(table, sd_idx, sums_out, cnt_out,
              sidx, idxg0, idxl0, idxg1, idxl1, idxlp, rows0, rows1, ones_v,
              cbuf, obuf, cnt_vm, sums_sh, cnt_sh, gsem0, gsem1, ssem0,
              ssem1):
    c = lax.axis_index("c")
    s = lax.axis_index("s")
    base_dst = c * PER_SC

    zeros16 = jnp.zeros((16,), jnp.float32)
    for j in range(2 * CHUNK // 16):
        ones_v[pl.ds(j * 16, 16)] = jnp.ones((16,), jnp.float32)

    def zrow_init(i, _):
        for j in range(D // 16):
            obuf[i, pl.ds(j * 16, 16)] = zeros16
        return _
    lax.fori_loop(0, FCHUNK, zrow_init, None)

    def cb_init(i, _):
        cbuf[pl.ds(i * 16, 16)] = zeros16
        return _
    lax.fori_loop(0, FCHUNK // 16, cb_init, None)
    if FCHUNK % 16:
        cbuf[pl.ds(FCHUNK - 16, 16)] = zeros16

    nk = (NFCHUNKS - s + NTILES - 1) // NTILES

    def zero_fire(t):
        r = pl.multiple_of((s + t * NTILES) * FCHUNK, 8)
        pltpu.async_copy(obuf, sums_sh.at[pl.ds(r, FCHUNK)], gsem0)
        pltpu.async_copy(cbuf, cnt_sh.at[pl.ds(r, FCHUNK)], gsem1)

    def zero_wait(t):
        r = pl.multiple_of((s + t * NTILES) * FCHUNK, 8)
        pltpu.make_async_copy(obuf, sums_sh.at[pl.ds(r, FCHUNK)],
                              gsem0).wait()
        pltpu.make_async_copy(cbuf, cnt_sh.at[pl.ds(r, FCHUNK)],
                              gsem1).wait()

    zero_fire(0)

    def zero_step(t, _):
        @pl.when(t + 1 < nk)
        def _():
            zero_fire(t + 1)
        zero_wait(t)
        return _
    lax.fori_loop(0, nk, zero_step, None)

    plsc.subcore_barrier()

    plo = (s * NPAIRS) // NTILES
    phi = ((s + 1) * NPAIRS) // NTILES

    def sidx_load(t):
        base = pl.multiple_of(t * (2 * CHUNK), 2 * CHUNK)
        pltpu.sync_copy(sd_idx.at[:, pl.ds(base, 2 * CHUNK)], sidx)

    def compute_idx(half, idxg, idxl):
        for j in range(CHUNK // 16):
            sl16 = pl.ds(half * CHUNK + j * 16, 16)
            sl = pl.ds(j * 16, 16)
            dl = sidx[1, sl16] - base_dst
            ok = (dl >= 0) & (dl < PER_SC)
            loc = jnp.where(ok, dl, -1)
            idxl[sl] = loc
            idxlp[sl16] = loc
            idxg[sl] = jnp.where(ok, sidx[0, sl16], -1)

    def scatter(rows, idxl):
        pltpu.sync_copy(rows, sums_sh.at[plsc.Indices(idxl, ignored_value=-1)],
                        add=True)
        pltpu.sync_copy(ones_v, cnt_sh.at[plsc.Indices(idxl, ignored_value=-1)],
                        add=True)

    def gather_start(idxg, rows, gsem):
        pltpu.async_copy(
            table.at[plsc.Indices(idxg, ignored_value=-1)], rows, gsem)

    def gather_wait(idxg, rows, gsem):
        pltpu.make_async_copy(
            table.at[plsc.Indices(idxg, ignored_value=-1)], rows, gsem).wait()

    def scatter_start(rows, idxl, ssem):
        pltpu.async_copy(
            rows, sums_sh.at[plsc.Indices(idxl, ignored_value=-1)], ssem,
            add=True)

    def scatter_wait(rows, idxl, ssem):
        pltpu.make_async_copy(
            rows, sums_sh.at[plsc.Indices(idxl, ignored_value=-1)],
            ssem).wait()

    def cnt_start():
        pltpu.async_copy(
            ones_v, cnt_sh.at[plsc.Indices(idxlp, ignored_value=-1)], ssem0,
            add=True)

    def cnt_wait():
        pltpu.make_async_copy(
            ones_v, cnt_sh.at[plsc.Indices(idxlp, ignored_value=-1)],
            ssem0).wait()

    sidx_load(plo)
    compute_idx(0, idxg0, idxl0)
    compute_idx(1, idxg1, idxl1)
    gather_start(idxg0, rows0, gsem0)
    cnt_start()

    def pair_step(t, _):
        @pl.when(t > plo)
        def _():
            scatter_wait(rows1, idxl1, ssem1)
            compute_idx(1, idxg1, idxl1)
            cnt_start()

        gather_wait(idxg0, rows0, gsem0)
        gather_start(idxg1, rows1, gsem1)
        scatter_start(rows0, idxl0, ssem0)
        scatter_wait(rows0, idxl0, ssem0)
        cnt_wait()

        @pl.when(t + 1 < phi)
        def _():
            sidx_load(t + 1)
            compute_idx(0, idxg0, idxl0)
            gather_start(idxg0, rows0, gsem0)
        gather_wait(idxg1, rows1, gsem1)
        scatter_start(rows1, idxl1, ssem1)
        return _

    lax.fori_loop(plo, phi, pair_step, None)
    scatter_wait(rows1, idxl1, ssem1)

    plsc.subcore_barrier()

    DUMP = 624
    r0 = pl.multiple_of(s * DUMP, 8)
    g0 = pl.multiple_of(base_dst + r0, 8)
    pltpu.sync_copy(sums_sh.at[pl.ds(r0, DUMP)],
                    sums_out.at[pl.ds(g0, DUMP)])
    pltpu.sync_copy(cnt_sh.at[pl.ds(r0, DUMP)], cnt_vm)
    pltpu.sync_copy(cnt_vm, cnt_out.at[pl.ds(g0, DUMP)])

    @pl.when(s == 0)
    def _tail():
        rt = pl.multiple_of(NTILES * DUMP, 8)
        gt = pl.multiple_of(base_dst + rt, 8)
        pltpu.sync_copy(sums_sh.at[pl.ds(rt, PER_SC - NTILES * DUMP)],
                        sums_out.at[pl.ds(gt, PER_SC - NTILES * DUMP)])
        pltpu.sync_copy(cnt_sh.at[pl.ds(rt, PER_SC - NTILES * DUMP)],
                        cnt_vm.at[pl.ds(0, PER_SC - NTILES * DUMP)])
        pltpu.sync_copy(cnt_vm.at[pl.ds(0, PER_SC - NTILES * DUMP)],
                        cnt_out.at[pl.ds(gt, PER_SC - NTILES * DUMP)])


@functools.partial(
    pl.kernel,
    out_type=(jax.ShapeDtypeStruct((N_DST, D), jnp.float32),
              jax.ShapeDtypeStruct((N_DST,), jnp.float32)),
    mesh=plsc.VectorSubcoreMesh(core_axis_name="c", subcore_axis_name="s"),
    scratch_types=[
        pltpu.VMEM((2, 2 * CHUNK), jnp.int32),
        pltpu.VMEM((CHUNK,), jnp.int32),
        pltpu.VMEM((CHUNK,), jnp.int32),
        pltpu.VMEM((CHUNK,), jnp.int32),
        pltpu.VMEM((CHUNK,), jnp.int32),
        pltpu.VMEM((2 * CHUNK,), jnp.int32),
        pltpu.VMEM((CHUNK, D), jnp.float32),
        pltpu.VMEM((CHUNK, D), jnp.float32),
        pltpu.VMEM((2 * CHUNK,), jnp.float32),
        pltpu.VMEM((FCHUNK,), jnp.float32),
        pltpu.VMEM((FCHUNK, D), jnp.float32),
        pltpu.VMEM((624,), jnp.float32),
        pltpu.VMEM_SHARED((PER_SC, D), jnp.float32),
        pltpu.VMEM_SHARED((PER_SC,), jnp.float32),
        pltpu.SemaphoreType.DMA,
        pltpu.SemaphoreType.DMA,
        pltpu.SemaphoreType.DMA,
        pltpu.SemaphoreType.DMA,
    ],
)
def _agg_call(table, sd_idx, sums_out, cnt_out, *scratch):
    _agg_body(table, sd_idx, sums_out, cnt_out, *scratch)


def _blend_body(z_ref, s_ref, c_ref, o_ref):
    o_ref[...] = (0.7 * z_ref[...]
                  + 0.3 * s_ref[...] / jnp.maximum(c_ref[...], 1.0))


BLK = 2000


def _blend(z, sums, cnt):
    return pl.pallas_call(
        _blend_body,
        out_shape=jax.ShapeDtypeStruct((N_DST, D), jnp.float32),
        grid=(N_DST // BLK,),
        in_specs=[
            pl.BlockSpec((BLK, D), lambda i: (i, 0)),
            pl.BlockSpec((BLK, D), lambda i: (i, 0)),
            pl.BlockSpec((BLK, 1), lambda i: (i, 0)),
        ],
        out_specs=pl.BlockSpec((BLK, D), lambda i: (i, 0)),
    )(z, sums, cnt.reshape(N_DST, 1))


@jax.jit
def _run(z_bill_version, z_bill, z_legislator_term, z_legislator,
         src_is_version, dst_is_version, src_same_person, dst_same_person):
    sd_b = jnp.stack([src_is_version, dst_is_version])
    sd_l = jnp.stack([src_same_person, dst_same_person])
    sums_b, cnt_b = _agg_call(z_bill_version, sd_b)
    sums_l, cnt_l = _agg_call(z_legislator_term, sd_l)
    out_b = _blend(z_bill, sums_b, cnt_b)
    out_l = _blend(z_legislator, sums_l, cnt_l)
    return out_b, out_l


def kernel(z_bill_version, z_bill, z_legislator_term, z_legislator,
           src_is_version, dst_is_version, src_same_person, dst_same_person):
    return _run(z_bill_version, z_bill, z_legislator_term, z_legislator,
                src_is_version, dst_is_version, src_same_person,
                dst_same_person)
